# baseline (speedup 1.0000x reference)
import functools

import jax
import jax.numpy as jnp
from jax import lax
from jax.experimental import pallas as pl
from jax.experimental.pallas import tpu as pltpu
from jax.experimental.pallas import tpu_sc as plsc

BATCH = 4096
DIM = 512
NUM_CLASSES = 10000
MARGIN = 0.35

B_TILE = 512
C_TILE = 2048
C_PAD = 10240
NB = BATCH // B_TILE
NC = C_PAD // C_TILE
N_PAD = float(C_PAD - NUM_CLASSES)

LANES = 128
NCHUNK = C_TILE // LANES

_SC_INFO = plsc.get_sparse_core_info()
SC_CORES = _SC_INFO.num_cores
SC_SUBCORES = _SC_INFO.num_subcores
SC_LANES = _SC_INFO.num_lanes
SC_WORKERS = SC_CORES * SC_SUBCORES
B_PER_W = BATCH // SC_WORKERS
HALF = B_PER_W // 2
D_CHUNKS = DIM // SC_LANES


def _norm_body(x_ref, o_ref):
    x = x_ref[...]
    o_ref[...] = x / (jnp.sqrt(jnp.sum(x * x, axis=1, keepdims=True)) + 1e-12)


def _row_normalize(x, row_tile):
    rows = x.shape[0]
    return pl.pallas_call(
        _norm_body,
        grid=(rows // row_tile,),
        in_specs=[pl.BlockSpec((row_tile, DIM), lambda i: (i, 0))],
        out_specs=pl.BlockSpec((row_tile, DIM), lambda i: (i, 0)),
        out_shape=jax.ShapeDtypeStruct(x.shape, jnp.float32),
    )(x)


def _sc_target_dot(fn, cn, label):
    mesh = plsc.VectorSubcoreMesh(core_axis_name="c", subcore_axis_name="s")

    @functools.partial(
        pl.kernel,
        mesh=mesh,
        out_type=jax.ShapeDtypeStruct((BATCH,), jnp.float32),
        scratch_types=[
            pltpu.VMEM((B_PER_W,), jnp.int32),
            pltpu.VMEM((HALF, DIM), jnp.float32),
            pltpu.VMEM((HALF, DIM), jnp.float32),
            pltpu.VMEM((B_PER_W,), jnp.float32),
            pltpu.SemaphoreType.DMA,
        ],
    )
    def sc_kernel(fn_hbm, cn_hbm, lbl_hbm, out_hbm,
                  idx_v, f_v, g_v, t_v, sem):
        wid = lax.axis_index("s") * SC_CORES + lax.axis_index("c")
        base = wid * B_PER_W
        lane_iota = lax.iota(jnp.int32, SC_LANES)
        pltpu.sync_copy(lbl_hbm.at[pl.ds(base, B_PER_W)], idx_v)
        for h in range(2):
            pltpu.async_copy(
                cn_hbm.at[idx_v.at[pl.ds(h * HALF, HALF)]], g_v, sem
            ).wait()
            pltpu.sync_copy(fn_hbm.at[pl.ds(base + h * HALF, HALF)], f_v)

            def group_dot(g, _):
                r0 = g * SC_LANES
                res = jnp.zeros((SC_LANES,), jnp.float32)
                for rr in range(SC_LANES):
                    r = r0 + rr
                    acc = f_v[r, pl.ds(0, SC_LANES)] * g_v[r, pl.ds(0, SC_LANES)]
                    for k in range(1, D_CHUNKS):
                        acc = acc + (
                            f_v[r, pl.ds(k * SC_LANES, SC_LANES)]
                            * g_v[r, pl.ds(k * SC_LANES, SC_LANES)]
                        )
                    for sh in (8, 4, 2, 1):
                        perm = (lane_iota + sh) & (SC_LANES - 1)
                        acc = acc + acc.at[perm].get(mode="promise_in_bounds")
                    res = jnp.where(lane_iota == rr, acc, res)
                t_v[pl.ds(h * HALF + r0, SC_LANES)] = res
                return _

            lax.fori_loop(0, HALF // SC_LANES, group_dot, 0)
        pltpu.sync_copy(t_v, out_hbm.at[pl.ds(base, B_PER_W)])

    return sc_kernel(fn, cn, label)


def _body(f_ref, c_ref, out_ref, s_scr):
    j = pl.program_id(0)
    i = pl.program_id(1)

    logits = jax.lax.dot_general(
        f_ref[...], c_ref[...], (((1,), (1,)), ((), ())),
        preferred_element_type=jnp.float32,
    )

    e = jnp.exp(logits)
    sum_e = e[:, :LANES]
    for k in range(1, NCHUNK):
        sum_e = sum_e + e[:, k * LANES:(k + 1) * LANES]

    @pl.when(j == 0)
    def _():
        s_scr[i] = sum_e

    @pl.when(j > 0)
    def _():
        s_scr[i] = s_scr[i] + sum_e

    @pl.when(j == NC - 1)
    def _():
        out_ref[0, :] = jnp.sum(s_scr[i], axis=1) - N_PAD


def _sum_exp(fn, cn):
    out = pl.pallas_call(
        _body,
        grid=(NC, NB),
        in_specs=[
            pl.BlockSpec((B_TILE, DIM), lambda j, i: (i, 0)),
            pl.BlockSpec((C_TILE, DIM), lambda j, i: (j, 0)),
        ],
        out_specs=pl.BlockSpec((1, B_TILE), lambda j, i: (0, i)),
        out_shape=jax.ShapeDtypeStruct((1, BATCH), jnp.float32),
        scratch_shapes=[
            pltpu.VMEM((NB, B_TILE, LANES), jnp.float32),
        ],
    )(fn, cn)
    return out


def _combine_body(s_ref, t_ref, o_ref):
    t = t_ref[...]
    tm = t - MARGIN
    s = s_ref[...] - jnp.exp(t) + jnp.exp(tm)
    o_ref[...] = jnp.log(s) - tm


def _combine(s, t):
    return pl.pallas_call(
        _combine_body,
        out_shape=jax.ShapeDtypeStruct((1, BATCH), jnp.float32),
    )(s, t)


def kernel(feature, label, centers):
    fn = _row_normalize(feature, 512)
    c_pad = jnp.pad(centers, ((0, C_PAD - NUM_CLASSES), (0, 0)))
    cn = _row_normalize(c_pad, 1024)

    t = _sc_target_dot(fn, cn, label)
    s = _sum_exp(fn, cn)
    out = _combine(s, t.reshape(1, BATCH))
    return out.reshape(BATCH)


# speedup vs baseline: 1.5289x; 1.5289x over previous
import jax
import jax.numpy as jnp
from jax.experimental import pallas as pl
from jax.experimental.pallas import tpu as pltpu

BATCH = 4096
DIM = 512
NUM_CLASSES = 10000
MARGIN = 0.35

B_TILE = 512
C_TILE = 2048
NB = BATCH // B_TILE
NC = -(-NUM_CLASSES // C_TILE)
LAST_VALID = NUM_CLASSES - (NC - 1) * C_TILE

LANES = 128
NCHUNK = C_TILE // LANES


def _rownorm(x):
    return x / (jnp.sqrt(jnp.sum(x * x, axis=1, keepdims=True)) + 1e-12)


def _chunk_sum(x):
    acc = x[:, :LANES]
    for k in range(1, NCHUNK):
        acc = acc + x[:, k * LANES:(k + 1) * LANES]
    return acc


def _body(f_ref, c_ref, lbl_ref, out_ref, fn_scr, cn_scr, s_scr, t_scr):
    j = pl.program_id(0)
    i = pl.program_id(1)

    @pl.when(j == 0)
    def _():
        fn_scr[i] = _rownorm(f_ref[...])

    @pl.when(i == 0)
    def _():
        cn_scr[...] = _rownorm(c_ref[...])

    logits = jax.lax.dot_general(
        fn_scr[i], cn_scr[...], (((1,), (1,)), ((), ())),
        preferred_element_type=jnp.float32,
    )

    e = jnp.exp(logits)
    lbl = lbl_ref[0, 0, :]
    cols = j * C_TILE + jax.lax.broadcasted_iota(jnp.int32, (B_TILE, C_TILE), 1)
    masked = jnp.where(cols == lbl[:, None], logits, 0.0)
    t_part = _chunk_sum(masked)

    @pl.when(j == 0)
    def _():
        s_scr[i] = _chunk_sum(e)
        t_scr[i] = t_part

    @pl.when(jnp.logical_and(j > 0, j < NC - 1))
    def _():
        s_scr[i] = s_scr[i] + _chunk_sum(e)
        t_scr[i] = t_scr[i] + t_part

    @pl.when(j == NC - 1)
    def _():
        lane = jax.lax.broadcasted_iota(jnp.int32, (B_TILE, C_TILE), 1)
        ee = jnp.where(lane < LAST_VALID, e, 0.0)
        s128 = s_scr[i] + _chunk_sum(ee)
        t = jnp.sum(t_scr[i] + t_part, axis=1)
        tm = t - MARGIN
        s = jnp.sum(s128, axis=1) - jnp.exp(t) + jnp.exp(tm)
        out_ref[0, :] = jnp.log(s) - tm


def kernel(feature, label, centers):
    lbl3 = label.reshape(NB, 1, B_TILE)
    out = pl.pallas_call(
        _body,
        grid=(NC, NB),
        in_specs=[
            pl.BlockSpec((B_TILE, DIM), lambda j, i: (i, 0)),
            pl.BlockSpec((C_TILE, DIM), lambda j, i: (j, 0)),
            pl.BlockSpec((1, 1, B_TILE), lambda j, i: (i, 0, 0)),
        ],
        out_specs=pl.BlockSpec((1, B_TILE), lambda j, i: (0, i)),
        out_shape=jax.ShapeDtypeStruct((1, BATCH), jnp.float32),
        scratch_shapes=[
            pltpu.VMEM((NB, B_TILE, DIM), jnp.float32),
            pltpu.VMEM((C_TILE, DIM), jnp.float32),
            pltpu.VMEM((NB, B_TILE, LANES), jnp.float32),
            pltpu.VMEM((NB, B_TILE, LANES), jnp.float32),
        ],
    )(feature, centers, lbl3)
    return out.reshape(BATCH)
